# baseline (speedup 1.0000x reference)
import jax
import jax.numpy as jnp
from jax.experimental import pallas as pl
from jax.experimental.pallas import tpu as pltpu

_TOTAL = 16384
_B = 16
_D_TXT = 1024
_D_LBL = 512
_N_LABELS = 64
_TILE = 2048
_GRID = _TOTAL // _TILE


def _body(x_ref, seg_ref, lbl_ref, lt_ref, w_ref, b_ref, out_ref,
          acc_ref, m_ref, d_ref):
    i = pl.program_id(0)

    @pl.when(i == 0)
    def _init():
        acc_ref[...] = jnp.zeros_like(acc_ref)
        m_ref[...] = jnp.full_like(m_ref, -jnp.inf)
        d_ref[...] = jnp.zeros_like(d_ref)

    x = x_ref[...]
    seg = seg_ref[...]
    lbl = lbl_ref[...]

    w_all = w_ref[...]
    w_lbl = w_all[0:_D_LBL, :]
    w_txt = w_all[_D_LBL:_D_LBL + _D_TXT, :]

    lbl_scores = jnp.dot(lt_ref[...], w_lbl,
                         preferred_element_type=jnp.float32)
    lf = (lbl == jax.lax.broadcasted_iota(jnp.int32, (_N_LABELS, _TILE), 0)
          ).astype(jnp.float32)
    s_lbl = jax.lax.dot_general(
        lbl_scores, lf, dimension_numbers=(((0,), (0,)), ((), ())),
        preferred_element_type=jnp.float32)

    s_txt = jax.lax.dot_general(
        w_txt, x, dimension_numbers=(((0,), (1,)), ((), ())),
        preferred_element_type=jnp.float32)
    s = s_txt + s_lbl + b_ref[0, 0]

    onehot = seg == jax.lax.broadcasted_iota(jnp.int32, (_B, _TILE), 0)
    of = onehot.astype(jnp.float32)

    tile_max = jnp.max(jnp.where(onehot, s, -jnp.inf),
                       axis=1, keepdims=True)
    m_old = m_ref[...]
    m_new = jnp.maximum(m_old, tile_max)
    rescale = jnp.where(m_old == -jnp.inf, 0.0, jnp.exp(m_old - m_new))
    m_ref[...] = m_new

    m_safe = jnp.where(m_new == -jnp.inf, 0.0, m_new)
    m_node = jnp.sum(of * m_safe, axis=0, keepdims=True)
    e = jnp.exp(s - m_node)
    oe = of * e

    d_ref[...] = d_ref[...] * rescale + jnp.sum(oe, axis=1, keepdims=True)
    contrib = jnp.dot(oe, x, preferred_element_type=jnp.float32)
    acc_ref[...] = acc_ref[...] * rescale + contrib

    @pl.when(i == _GRID - 1)
    def _finish():
        out_ref[...] = acc_ref[...] / (d_ref[...] + 1e-9)


def kernel(node_embedding, label_ids, segment_ids, label_table, W, b):
    seg3 = segment_ids.astype(jnp.int32).reshape(_GRID, 1, _TILE)
    lbl3 = label_ids.astype(jnp.int32).reshape(_GRID, 1, _TILE)
    b2 = b.reshape(1, 1)

    out = pl.pallas_call(
        _body,
        grid=(_GRID,),
        in_specs=[
            pl.BlockSpec((_TILE, _D_TXT), lambda i: (i, 0)),
            pl.BlockSpec((None, 1, _TILE), lambda i: (i, 0, 0)),
            pl.BlockSpec((None, 1, _TILE), lambda i: (i, 0, 0)),
            pl.BlockSpec((_N_LABELS, _D_LBL), lambda i: (0, 0)),
            pl.BlockSpec((_D_LBL + _D_TXT, 1), lambda i: (0, 0)),
            pl.BlockSpec((1, 1), lambda i: (0, 0)),
        ],
        out_specs=pl.BlockSpec((_B, _D_TXT), lambda i: (0, 0)),
        out_shape=jax.ShapeDtypeStruct((_B, _D_TXT), jnp.float32),
        scratch_shapes=[
            pltpu.VMEM((_B, _D_TXT), jnp.float32),
            pltpu.VMEM((_B, 1), jnp.float32),
            pltpu.VMEM((_B, 1), jnp.float32),
        ],
    )(node_embedding, seg3, lbl3, label_table, W, b2)
    return out


# speedup vs baseline: 1.0353x; 1.0353x over previous
import jax
import jax.numpy as jnp
from jax.experimental import pallas as pl
from jax.experimental.pallas import tpu as pltpu

_TOTAL = 16384
_B = 16
_D_TXT = 1024
_D_LBL = 512
_N_LABELS = 64
_TILE = 4096
_GRID = _TOTAL // _TILE


def _body(x_ref, seg_ref, lbl_ref, lt_ref, w_ref, b_ref, out_ref,
          acc_ref, m_ref, d_ref):
    i = pl.program_id(0)

    @pl.when(i == 0)
    def _init():
        acc_ref[...] = jnp.zeros_like(acc_ref)
        m_ref[...] = jnp.full_like(m_ref, -jnp.inf)
        d_ref[...] = jnp.zeros_like(d_ref)

    x = x_ref[...]
    seg = seg_ref[...]
    lbl = lbl_ref[...]

    w_all = w_ref[...]
    w_lbl = w_all[0:_D_LBL, :]
    w_txt = w_all[_D_LBL:_D_LBL + _D_TXT, :]

    lbl_scores = jnp.dot(lt_ref[...], w_lbl,
                         preferred_element_type=jnp.float32)
    lf = (lbl == jax.lax.broadcasted_iota(jnp.int32, (_N_LABELS, _TILE), 0)
          ).astype(jnp.float32)
    s_lbl = jax.lax.dot_general(
        lbl_scores, lf, dimension_numbers=(((0,), (0,)), ((), ())),
        preferred_element_type=jnp.float32)

    s_txt = jax.lax.dot_general(
        w_txt, x, dimension_numbers=(((0,), (1,)), ((), ())),
        preferred_element_type=jnp.float32)
    s = s_txt + s_lbl + b_ref[0, 0]

    onehot = seg == jax.lax.broadcasted_iota(jnp.int32, (_B, _TILE), 0)
    of = onehot.astype(jnp.float32)

    tile_max = jnp.max(jnp.where(onehot, s, -jnp.inf),
                       axis=1, keepdims=True)
    m_old = m_ref[...]
    m_new = jnp.maximum(m_old, tile_max)
    rescale = jnp.where(m_old == -jnp.inf, 0.0, jnp.exp(m_old - m_new))
    m_ref[...] = m_new

    m_safe = jnp.where(m_new == -jnp.inf, 0.0, m_new)
    m_node = jnp.sum(of * m_safe, axis=0, keepdims=True)
    e = jnp.exp(s - m_node)
    oe = of * e

    d_ref[...] = d_ref[...] * rescale + jnp.sum(oe, axis=1, keepdims=True)
    contrib = jnp.dot(oe, x, preferred_element_type=jnp.float32)
    acc_ref[...] = acc_ref[...] * rescale + contrib

    @pl.when(i == _GRID - 1)
    def _finish():
        out_ref[...] = acc_ref[...] / (d_ref[...] + 1e-9)


def kernel(node_embedding, label_ids, segment_ids, label_table, W, b):
    seg3 = segment_ids.astype(jnp.int32).reshape(_GRID, 1, _TILE)
    lbl3 = label_ids.astype(jnp.int32).reshape(_GRID, 1, _TILE)
    b2 = b.reshape(1, 1)

    out = pl.pallas_call(
        _body,
        grid=(_GRID,),
        in_specs=[
            pl.BlockSpec((_TILE, _D_TXT), lambda i: (i, 0)),
            pl.BlockSpec((None, 1, _TILE), lambda i: (i, 0, 0)),
            pl.BlockSpec((None, 1, _TILE), lambda i: (i, 0, 0)),
            pl.BlockSpec((_N_LABELS, _D_LBL), lambda i: (0, 0)),
            pl.BlockSpec((_D_LBL + _D_TXT, 1), lambda i: (0, 0)),
            pl.BlockSpec((1, 1), lambda i: (0, 0)),
        ],
        out_specs=pl.BlockSpec((_B, _D_TXT), lambda i: (0, 0)),
        out_shape=jax.ShapeDtypeStruct((_B, _D_TXT), jnp.float32),
        scratch_shapes=[
            pltpu.VMEM((_B, _D_TXT), jnp.float32),
            pltpu.VMEM((_B, 1), jnp.float32),
            pltpu.VMEM((_B, 1), jnp.float32),
        ],
    )(node_embedding, seg3, lbl3, label_table, W, b2)
    return out
